# baseline (speedup 1.0000x reference)
import functools

import jax
import jax.numpy as jnp
from jax import lax
from jax.experimental import pallas as pl
from jax.experimental.pallas import tpu as pltpu
from jax.experimental.pallas import tpu_sc as plsc

_B, _P, _C, _O = 32, 8732, 81, 12
_NEG_POS = 3
_OVTH = 0.5
_PB = 8736
_CEW = 2048
_L = 16
_CH = _PB // _L
_ONEBITS = 0x3F800000
_LC = (0.043432123558091715, -0.40489056137051477, 1.593968600299121,
       -3.492589774569492, 5.046943036625113, -2.786831523366335)
_LN2 = 0.6931471805599453


def _match_body(gt_ref, pri_ref, locp_ref, conf_ref, lp_ref, np_ref):
    px = pri_ref[0:1, :]
    py = pri_ref[1:2, :]
    pw = pri_ref[2:3, :]
    ph = pri_ref[3:4, :]
    x1 = px - pw * 0.5
    y1 = py - ph * 0.5
    x2 = px + pw * 0.5
    y2 = py + ph * 0.5
    area_p = (x2 - x1) * (y2 - y1)
    g = gt_ref[0]
    tx1 = g[:, 0:1]
    ty1 = g[:, 1:2]
    tx2 = g[:, 2:3]
    ty2 = g[:, 3:4]
    lab = g[:, 4:5]
    ix = jnp.maximum(jnp.minimum(tx2, x2) - jnp.maximum(tx1, x1), 0.0)
    iy = jnp.maximum(jnp.minimum(ty2, y2) - jnp.maximum(ty1, y1), 0.0)
    inter = ix * iy
    area_t = (tx2 - tx1) * (ty2 - ty1)
    ov = inter / (area_t + area_p - inter)

    tio = jax.lax.broadcasted_iota(jnp.int32, ov.shape, 0)
    lio = jax.lax.broadcasted_iota(jnp.int32, ov.shape, 1)
    bov = jnp.max(ov, axis=0, keepdims=True)
    bidx = jnp.min(jnp.where(ov == bov, tio, _O), axis=0, keepdims=True)
    rm = jnp.max(ov, axis=1, keepdims=True)
    bpi = jnp.min(jnp.where(ov == rm, lio, _PB), axis=1, keepdims=True)
    hit = lio == bpi
    last_t = jnp.max(jnp.where(hit, tio, -1), axis=0, keepdims=True)
    anyh = last_t >= 0
    bov = jnp.where(anyh, 2.0, bov)
    bidx = jnp.where(anyh, last_t, bidx)

    sel = bidx == tio
    conf = jnp.sum(jnp.where(sel, lab, 0.0), axis=0, keepdims=True)
    mx1 = jnp.sum(jnp.where(sel, tx1, 0.0), axis=0, keepdims=True)
    my1 = jnp.sum(jnp.where(sel, ty1, 0.0), axis=0, keepdims=True)
    mx2 = jnp.sum(jnp.where(sel, tx2, 0.0), axis=0, keepdims=True)
    my2 = jnp.sum(jnp.where(sel, ty2, 0.0), axis=0, keepdims=True)
    confi = jnp.where(bov < _OVTH, 0, conf.astype(jnp.int32))
    conf_ref[...] = confi.reshape(1, 1, _PB)
    posf = (confi > 0).astype(jnp.float32)

    g_cx = ((mx1 + mx2) * 0.5 - px) / (0.1 * pw)
    g_cy = ((my1 + my2) * 0.5 - py) / (0.1 * ph)
    g_w = jnp.log((mx2 - mx1) / pw) / 0.2
    g_h = jnp.log((my2 - my1) / ph) / 0.2
    lp4 = locp_ref[0]
    s = jnp.float32(0.0)
    for d in range(4):
        df = lp4[d:d + 1, :] - (g_cx, g_cy, g_w, g_h)[d]
        ad = jnp.abs(df)
        s = s + jnp.sum(jnp.where(ad < 1.0, 0.5 * df * df, ad - 0.5) * posf)
    lp_ref[...] = s.reshape(1, 1, 1)
    np_ref[...] = jnp.sum(posf).astype(jnp.int32).reshape(1, 1, 1)


def _ce_body(x_ref, tgt_ref, out_ref):
    xt = x_ref[0].T
    e = jnp.exp(xt)
    s = jnp.sum(e, axis=0, keepdims=True)
    cls = jax.lax.broadcasted_iota(jnp.int32, xt.shape, 0)
    tgt = tgt_ref[0]
    eg = jnp.sum(jnp.where(cls == tgt, e, 0.0), axis=0, keepdims=True)
    out_ref[0] = s / eg


def _sel_body(ratio_ref, conf_ref, lp_ref, loc_out, conf_out):
    r = ratio_ref[...]
    ce = jnp.log(r)
    pos = conf_ref[...] > 0
    posf = pos.astype(jnp.float32)
    nposi = jnp.sum(pos.astype(jnp.int32), axis=1, keepdims=True)
    total = jnp.sum(nposi)
    k = jnp.minimum(_NEG_POS * nposi, _P - total - 1)
    cl = jnp.where(pos, 1.0, r)
    clb = jax.lax.bitcast_convert_type(cl, jnp.int32)
    lo = jnp.zeros((_B, 1), jnp.int32)
    for bit in range(30, -1, -1):
        cand = lo + (1 << bit)
        cnt = jnp.sum((clb >= cand).astype(jnp.int32), axis=1, keepdims=True)
        lo = jnp.where(cnt >= k, cand, lo)
    gt = clb > lo
    eq = clb == lo
    cnt_gt = jnp.sum(gt.astype(jnp.float32), axis=1, keepdims=True)
    sum_gt = jnp.sum(jnp.where(gt, ce, 0.0), axis=1, keepdims=True)
    cnt_eq = jnp.sum(eq.astype(jnp.float32), axis=1, keepdims=True)
    sum_eq = jnp.sum(jnp.where(eq, ce, 0.0), axis=1, keepdims=True)
    kf = k.astype(jnp.float32)
    neg_sum = sum_gt + (kf - cnt_gt) * sum_eq / jnp.maximum(cnt_eq, 1.0)
    pos_sum = jnp.sum(ce * posf)
    nm = jnp.sum(posf)
    loc_out[...] = (jnp.sum(lp_ref[...]) / nm).reshape(1, 1)
    conf_out[...] = ((pos_sum + jnp.sum(neg_sum)) / nm).reshape(1, 1)


def _sc_select_body(ratio_hbm, conf_hbm, npos_hbm, out_hbm, rv, cv, bwv, npv, outv):
    cid = lax.axis_index("c")
    sid = lax.axis_index("s")
    w = sid * 2 + cid
    pltpu.sync_copy(ratio_hbm.at[w], rv)
    pltpu.sync_copy(conf_hbm.at[w], cv)
    pltpu.sync_copy(npos_hbm, npv)

    lane = lax.broadcasted_iota(jnp.int32, (_L,), 0)
    n0 = npv[pl.ds(0, _L)]
    n1 = npv[pl.ds(_L, _L)]
    total = jnp.sum(n0) + jnp.sum(n1)
    myn = (jnp.sum(jnp.where(lane == w, n0, 0))
           + jnp.sum(jnp.where(lane == w - _L, n1, 0)))
    k = jnp.minimum(_NEG_POS * myn, _P - total - 1)

    def p1(i, c):
        r = rv[pl.ds(i * _L, _L)]
        cf = cv[pl.ds(i * _L, _L)]
        b = lax.bitcast_convert_type(r, jnp.int32)
        bwv[pl.ds(i * _L, _L)] = jnp.where(cf > 0, _ONEBITS, b)
        return c
    lax.fori_loop(0, _CH, p1, 0)

    def sbit(i, lo):
        cand = lo + (1 << (30 - i))
        def cbody(j, acc):
            bb = bwv[pl.ds(j * _L, _L)]
            return acc + jnp.where(bb >= cand, 1, 0)
        cnt = jnp.sum(lax.fori_loop(0, _CH, cbody, jnp.zeros((_L,), jnp.int32)))
        return jnp.where(cnt >= k, cand, lo)
    lo = lax.fori_loop(0, 31, sbit, jnp.int32(0))

    def fin(j, carry):
        sgt, cgt, seq, ceq, psum = carry
        r = rv[pl.ds(j * _L, _L)]
        cf = cv[pl.ds(j * _L, _L)]
        bb = bwv[pl.ds(j * _L, _L)]
        bi = lax.bitcast_convert_type(r, jnp.int32)
        ex = lax.shift_right_arithmetic(bi, 23) - 127
        m = lax.bitcast_convert_type((bi & 0x7FFFFF) | _ONEBITS, jnp.float32)
        p = _LC[0]
        for cco in _LC[1:]:
            p = p * m + cco
        ce = (ex.astype(jnp.float32) + p) * _LN2
        gt = bb > lo
        eq = bb == lo
        return (sgt + jnp.where(gt, ce, 0.0),
                cgt + jnp.where(gt, 1, 0),
                seq + jnp.where(eq, ce, 0.0),
                ceq + jnp.where(eq, 1, 0),
                psum + jnp.where(cf > 0, ce, 0.0))
    z = jnp.zeros((_L,), jnp.float32)
    zi = jnp.zeros((_L,), jnp.int32)
    sgt, cgt, seq, ceq, psum = lax.fori_loop(0, _CH, fin, (z, zi, z, zi, z))
    cnt_eq = jnp.maximum(jnp.sum(ceq), 1)
    scale = (k - jnp.sum(cgt)).astype(jnp.float32) * jnp.sum(seq)
    extra_v = (jnp.full((_L,), scale, jnp.float32)
               / jnp.full((_L,), cnt_eq.astype(jnp.float32), jnp.float32))
    base = jnp.sum(psum) + jnp.sum(sgt)
    outv[...] = jnp.full((_L,), base, jnp.float32) + extra_v
    pltpu.sync_copy(outv, out_hbm.at[w])


_sc_select = functools.partial(
    pl.kernel,
    mesh=plsc.VectorSubcoreMesh(core_axis_name="c", subcore_axis_name="s"),
    compiler_params=pltpu.CompilerParams(needs_layout_passes=False),
    out_type=jax.ShapeDtypeStruct((_B, _L), jnp.float32),
    scratch_types=[
        pltpu.VMEM((_PB,), jnp.float32),
        pltpu.VMEM((_PB,), jnp.int32),
        pltpu.VMEM((_PB,), jnp.int32),
        pltpu.VMEM((_B,), jnp.int32),
        pltpu.VMEM((_L,), jnp.float32),
    ],
)(_sc_select_body)


def _comb_body(lp_ref, np_ref, parts_ref, loc_out, conf_out):
    nm = jnp.sum(np_ref[...]).astype(jnp.float32)
    loc_out[...] = (jnp.sum(lp_ref[...]) / nm).reshape(1, 1)
    conf_out[...] = (jnp.sum(parts_ref[:, 0:1]) / nm).reshape(1, 1)


@jax.jit
def kernel(loc_preds, score_preds, gt_data, priors):
    pad = jnp.tile(jnp.array([[2.0, 2.0, 1.0, 1.0]], jnp.float32),
                   (_PB - _P, 1))
    pri2 = jnp.concatenate([priors, pad], axis=0).T
    locp_t = jnp.transpose(loc_preds, (0, 2, 1))
    locp2 = jnp.pad(locp_t, ((0, 0), (0, 0), (0, _PB - _P)))

    conf8, lp, npos = pl.pallas_call(
        _match_body,
        grid=(_B,),
        in_specs=[
            pl.BlockSpec((1, _O, 5), lambda b: (b, 0, 0)),
            pl.BlockSpec((4, _PB), lambda b: (0, 0)),
            pl.BlockSpec((1, 4, _PB), lambda b: (b, 0, 0)),
        ],
        out_specs=[
            pl.BlockSpec((1, 1, _PB), lambda b: (b, 0, 0)),
            pl.BlockSpec((1, 1, 1), lambda b: (b, 0, 0)),
            pl.BlockSpec((1, 1, 1), lambda b: (b, 0, 0)),
        ],
        out_shape=[
            jax.ShapeDtypeStruct((_B, 1, _PB), jnp.int32),
            jax.ShapeDtypeStruct((_B, 1, 1), jnp.float32),
            jax.ShapeDtypeStruct((_B, 1, 1), jnp.int32),
        ],
    )(gt_data, pri2, locp2)
    conf = conf8.reshape(_B, _PB)[:, :_P]

    nblk = pl.cdiv(_P, _CEW)
    ratio = pl.pallas_call(
        _ce_body,
        grid=(_B, nblk),
        in_specs=[
            pl.BlockSpec((1, _CEW, _C), lambda b, i: (b, i, 0)),
            pl.BlockSpec((1, 1, _CEW), lambda b, i: (b, 0, i)),
        ],
        out_specs=pl.BlockSpec((1, 1, _CEW), lambda b, i: (b, 0, i)),
        out_shape=jax.ShapeDtypeStruct((_B, 1, _P), jnp.float32),
    )(score_preds, conf.reshape(_B, 1, _P))

    ratio_pad = jnp.pad(ratio.reshape(_B, _P), ((0, 0), (0, _PB - _P)),
                        constant_values=1.0)
    parts = _sc_select(ratio_pad, conf8.reshape(_B, _PB),
                       npos.reshape(_B))
    loc_l, conf_l = pl.pallas_call(
        _comb_body,
        out_shape=[
            jax.ShapeDtypeStruct((1, 1), jnp.float32),
            jax.ShapeDtypeStruct((1, 1), jnp.float32),
        ],
    )(lp.reshape(_B, 1), npos.reshape(_B, 1), parts)
    return (loc_l[0, 0], conf_l[0, 0])
